# baseline (speedup 1.0000x reference)
import functools

import jax
import jax.numpy as jnp
from jax import lax
from jax.experimental import pallas as pl
from jax.experimental.pallas import tpu as pltpu
from jax.experimental.pallas import tpu_sc as plsc

NC, NS, L = 2, 16, 16
NW = NC * NS
CHUNK = 128
DEGW = 8


def _sc_aggregate(x, src3, dst3, n_nodes, d_feat, n_chunks):
    rpt = (n_nodes // NS) & ~7
    tail = n_nodes - NS * rpt
    assert tail % 8 == 0 and tail <= rpt
    nacc = n_nodes + 8
    mesh = plsc.VectorSubcoreMesh(core_axis_name="c", subcore_axis_name="s")

    zacc = jnp.zeros((rpt, d_feat), jnp.float32)
    zdeg = jnp.zeros((rpt, DEGW), jnp.float32)
    ones8 = jnp.ones((CHUNK, DEGW), jnp.float32)

    @functools.partial(
        pl.kernel,
        mesh=mesh,
        out_type=[
            pltpu.HBM((NC, n_nodes, d_feat), jnp.float32),
            pltpu.HBM((NC, n_nodes, DEGW), jnp.float32),
        ],
        scratch_types=[
            pltpu.VMEM((2, CHUNK), jnp.int32),
            pltpu.VMEM((2, CHUNK), jnp.int32),
            pltpu.VMEM((2, CHUNK, d_feat), jnp.float32),
            pltpu.VMEM((CHUNK, DEGW), jnp.float32),
            pltpu.VMEM_SHARED((nacc, d_feat), jnp.float32),
            pltpu.VMEM_SHARED((nacc, DEGW), jnp.float32),
            pltpu.SemaphoreType.DMA((2,)),
            pltpu.SemaphoreType.DMA,
        ],
        compiler_params=pltpu.CompilerParams(use_tc_tiling_on_sc=False),
    )
    def agg_kernel(x_hbm, src_hbm, dst_hbm, zacc_hbm, zdeg_hbm, ones_hbm,
                   acc_out, deg_out,
                   src_v, dst_v, rows_v, ones_v, acc_s, deg_s, gsem, isem):
        cid = lax.axis_index("c")
        sid = lax.axis_index("s")
        wid = sid * NC + cid
        base = sid * rpt

        pltpu.sync_copy(ones_hbm, ones_v)

        pltpu.sync_copy(zacc_hbm, acc_s.at[pl.ds(base, rpt)])
        pltpu.sync_copy(zdeg_hbm, deg_s.at[pl.ds(base, rpt)])

        @pl.when(sid == NS - 1)
        def _zero_tail():
            pltpu.sync_copy(zacc_hbm.at[pl.ds(0, tail + 8)],
                            acc_s.at[pl.ds(NS * rpt, tail + 8)])
            pltpu.sync_copy(zdeg_hbm.at[pl.ds(0, tail + 8)],
                            deg_s.at[pl.ds(NS * rpt, tail + 8)])
        plsc.subcore_barrier()

        pltpu.sync_copy(src_hbm.at[wid, 0], src_v.at[0])
        pltpu.sync_copy(dst_hbm.at[wid, 0], dst_v.at[0])
        pltpu.async_copy(src_hbm.at[wid, 1], src_v.at[1], isem)
        pltpu.async_copy(dst_hbm.at[wid, 1], dst_v.at[1], isem)
        pltpu.async_copy(x_hbm.at[src_v.at[0]], rows_v.at[0], gsem.at[0])

        def chunk_body(j, _):
            p = lax.rem(j, 2)
            q = 1 - p

            @pl.when(j + 1 < n_chunks)
            def _issue_next_gather():
                pltpu.make_async_copy(src_hbm.at[wid, j + 1],
                                      src_v.at[q], isem).wait()
                pltpu.make_async_copy(dst_hbm.at[wid, j + 1],
                                      dst_v.at[q], isem).wait()
                pltpu.async_copy(x_hbm.at[src_v.at[q]], rows_v.at[q],
                                 gsem.at[q])

            pltpu.make_async_copy(x_hbm.at[src_v.at[p]], rows_v.at[p],
                                  gsem.at[p]).wait()
            pltpu.sync_copy(rows_v.at[p], acc_s.at[dst_v.at[p]], add=True)
            pltpu.sync_copy(ones_v, deg_s.at[dst_v.at[p]], add=True)

            @pl.when(j + 2 < n_chunks)
            def _prefetch_ids():
                pltpu.async_copy(src_hbm.at[wid, j + 2], src_v.at[p], isem)
                pltpu.async_copy(dst_hbm.at[wid, j + 2], dst_v.at[p], isem)
            return 0
        lax.fori_loop(0, n_chunks, chunk_body, 0)
        plsc.subcore_barrier()

        pltpu.sync_copy(acc_s.at[pl.ds(base, rpt)],
                        acc_out.at[cid, pl.ds(base, rpt)])
        pltpu.sync_copy(deg_s.at[pl.ds(base, rpt)],
                        deg_out.at[cid, pl.ds(base, rpt)])

        @pl.when(sid == NS - 1)
        def _read_tail():
            pltpu.sync_copy(acc_s.at[pl.ds(NS * rpt, tail)],
                            acc_out.at[cid, pl.ds(NS * rpt, tail)])
            pltpu.sync_copy(deg_s.at[pl.ds(NS * rpt, tail)],
                            deg_out.at[cid, pl.ds(NS * rpt, tail)])

    return agg_kernel(x, src3, dst3, zacc, zdeg, ones8)


def _tc_fused(acc2, deg2, x, batch3, wcat, bl2, wlin_t, blin2,
              n_nodes, d_feat, h_feat, f_out, n_graphs, blk):
    nsteps = n_nodes // blk
    two_h = 2 * h_feat

    def body(acc_ref, deg_ref, x_ref, batch_ref, wcat_ref, bl_ref,
             wlin_ref, blin_ref, out_ref, max_s, sum_s, cnt_s):
        i = pl.program_id(0)

        @pl.when(i == 0)
        def _init():
            max_s[...] = jnp.full((n_graphs, h_feat), -jnp.inf, jnp.float32)
            sum_s[...] = jnp.zeros((n_graphs, h_feat), jnp.float32)
            cnt_s[...] = jnp.zeros((n_graphs, 1), jnp.float32)

        a = acc_ref[0] + acc_ref[1]
        d = deg_ref[0, :, 0:1] + deg_ref[1, :, 0:1]
        agg = a / jnp.maximum(d, 1.0)
        cat = jnp.concatenate([agg, x_ref[...]], axis=1)
        h = lax.dot_general(cat, wcat_ref[...], (((1,), (0,)), ((), ())),
                            preferred_element_type=jnp.float32)
        h = jnp.maximum(h + bl_ref[...], 0.0)

        b2 = batch_ref[0]
        gids = lax.broadcasted_iota(jnp.int32, (1, n_graphs), 1)
        onehot = (b2 == gids).astype(jnp.float32)
        sum_s[...] += lax.dot_general(onehot, h, (((0,), (0,)), ((), ())),
                                      preferred_element_type=jnp.float32)
        ones_col = jnp.ones((blk, 1), jnp.float32)
        cnt_s[...] += lax.dot_general(onehot, ones_col,
                                      (((0,), (0,)), ((), ())),
                                      preferred_element_type=jnp.float32)

        for g in range(n_graphs):
            mg = b2 == g

            @pl.when(jnp.any(mg))
            def _masked_max(mg=mg):
                hb = jnp.where(mg, h, -jnp.inf)
                mx = jnp.max(hb, axis=0, keepdims=True)
                max_s[g:g + 1, :] = jnp.maximum(max_s[g:g + 1, :], mx)

        @pl.when(i == nsteps - 1)
        def _final():
            xm = max_s[...]
            xm = jnp.where(jnp.isfinite(xm), xm, 0.0)
            mean = sum_s[...] / jnp.maximum(cnt_s[...], 1.0)
            pooled = jnp.concatenate([xm, mean], axis=1)
            out_ref[...] = lax.dot_general(
                pooled, wlin_ref[...], (((1,), (0,)), ((), ())),
                preferred_element_type=jnp.float32) + blin_ref[...]

    return pl.pallas_call(
        body,
        grid=(nsteps,),
        in_specs=[
            pl.BlockSpec((NC, blk, d_feat), lambda i: (0, i, 0)),
            pl.BlockSpec((NC, blk, DEGW), lambda i: (0, i, 0)),
            pl.BlockSpec((blk, d_feat), lambda i: (i, 0)),
            pl.BlockSpec((1, blk, 1), lambda i: (i, 0, 0)),
            pl.BlockSpec((two_h, h_feat), lambda i: (0, 0)),
            pl.BlockSpec((1, h_feat), lambda i: (0, 0)),
            pl.BlockSpec((two_h, f_out), lambda i: (0, 0)),
            pl.BlockSpec((1, f_out), lambda i: (0, 0)),
        ],
        out_specs=pl.BlockSpec((n_graphs, f_out), lambda i: (0, 0)),
        out_shape=jax.ShapeDtypeStruct((n_graphs, f_out), jnp.float32),
        scratch_shapes=[
            pltpu.VMEM((n_graphs, h_feat), jnp.float32),
            pltpu.VMEM((n_graphs, h_feat), jnp.float32),
            pltpu.VMEM((n_graphs, 1), jnp.float32),
        ],
        compiler_params=pltpu.CompilerParams(
            dimension_semantics=("arbitrary",)),
    )(acc2, deg2, x, batch3, wcat, bl2, wlin_t, blin2)


def kernel(x, edge_index, batch, W_l, b_l, W_r, W_lin, b_lin):
    n_nodes, d_feat = x.shape
    n_edges = edge_index.shape[1]
    h_feat = W_l.shape[0]
    f_out = W_lin.shape[0]
    n_graphs = 64

    ept = n_edges // NW
    ept_pad = ((ept + CHUNK - 1) // CHUNK) * CHUNK
    n_chunks = ept_pad // CHUNK
    ei = edge_index.reshape(2, NW, ept)
    pad_src = jnp.zeros((NW, ept_pad - ept), jnp.int32)
    pad_dst = jnp.full((NW, ept_pad - ept), n_nodes, jnp.int32)
    src3 = jnp.concatenate([ei[0], pad_src], axis=1).reshape(
        NW, n_chunks, CHUNK)
    dst3 = jnp.concatenate([ei[1], pad_dst], axis=1).reshape(
        NW, n_chunks, CHUNK)

    acc2, deg2 = _sc_aggregate(x, src3, dst3, n_nodes, d_feat, n_chunks)

    blk = 1000
    batch3 = batch.astype(jnp.int32).reshape(n_nodes // blk, blk, 1)
    wcat = jnp.concatenate([W_l, W_r], axis=1).T
    wlin_t = W_lin.T
    bl2 = b_l.reshape(1, h_feat)
    blin2 = b_lin.reshape(1, f_out)

    return _tc_fused(acc2, deg2, x, batch3, wcat, bl2, wlin_t, blin2,
                     n_nodes, d_feat, h_feat, f_out, n_graphs, blk)


# speedup vs baseline: 1.7106x; 1.7106x over previous
import functools

import jax
import jax.numpy as jnp
from jax import lax
from jax.experimental import pallas as pl
from jax.experimental.pallas import tpu as pltpu
from jax.experimental.pallas import tpu_sc as plsc

NC, NS, L = 2, 16, 16
NW = NC * NS
CHUNK = 125
DEGW = 8
IB = 8


def _sc_aggregate(x, src3, dst3, n_nodes, d_feat, n_chunks):
    rpt = (n_nodes // NS) & ~7
    tail = n_nodes - NS * rpt
    assert tail % 8 == 0 and tail <= rpt
    assert n_chunks % IB == 0
    mesh = plsc.VectorSubcoreMesh(core_axis_name="c", subcore_axis_name="s")

    zacc = jnp.zeros((rpt, d_feat), jnp.float32)
    zdeg = jnp.zeros((rpt, DEGW), jnp.float32)
    ones8 = jnp.ones((CHUNK, DEGW), jnp.float32)

    @functools.partial(
        pl.kernel,
        mesh=mesh,
        out_type=[
            pltpu.HBM((NC, n_nodes, d_feat), jnp.float32),
            pltpu.HBM((NC, n_nodes, DEGW), jnp.float32),
        ],
        scratch_types=[
            pltpu.VMEM((IB, CHUNK), jnp.int32),
            pltpu.VMEM((IB, CHUNK), jnp.int32),
            pltpu.VMEM((CHUNK, d_feat), jnp.float32),
            pltpu.VMEM((CHUNK, d_feat), jnp.float32),
            pltpu.VMEM((CHUNK, DEGW), jnp.float32),
            pltpu.VMEM_SHARED((n_nodes, d_feat), jnp.float32),
            pltpu.VMEM_SHARED((n_nodes, DEGW), jnp.float32),
            pltpu.SemaphoreType.DMA,
            pltpu.SemaphoreType.DMA,
        ],
        compiler_params=pltpu.CompilerParams(use_tc_tiling_on_sc=False),
    )
    def agg_kernel(x_hbm, src_hbm, dst_hbm, zacc_hbm, zdeg_hbm, ones_hbm,
                   acc_out, deg_out,
                   src_v, dst_v, rows_a, rows_b, ones_v, acc_s, deg_s,
                   sem_a, sem_b):
        cid = lax.axis_index("c")
        sid = lax.axis_index("s")
        wid = sid * NC + cid
        base = sid * rpt

        pltpu.sync_copy(ones_hbm, ones_v)

        pltpu.sync_copy(zacc_hbm, acc_s.at[pl.ds(base, rpt)])
        pltpu.sync_copy(zdeg_hbm, deg_s.at[pl.ds(base, rpt)])

        @pl.when(sid == NS - 1)
        def _zero_tail():
            pltpu.sync_copy(zacc_hbm.at[pl.ds(0, tail)],
                            acc_s.at[pl.ds(NS * rpt, tail)])
            pltpu.sync_copy(zdeg_hbm.at[pl.ds(0, tail)],
                            deg_s.at[pl.ds(NS * rpt, tail)])
        plsc.subcore_barrier()

        bufs = [(rows_a, sem_a), (rows_b, sem_b)]

        def block_body(b, _):
            pltpu.sync_copy(src_hbm.at[wid, pl.ds(b * IB, IB)], src_v)
            pltpu.sync_copy(dst_hbm.at[wid, pl.ds(b * IB, IB)], dst_v)

            pltpu.async_copy(x_hbm.at[src_v.at[0]], rows_a, sem_a)
            for o in range(IB):
                rows_c, sem_c = bufs[o % 2]
                rows_n, sem_n = bufs[(o + 1) % 2]
                if o + 1 < IB:
                    pltpu.async_copy(x_hbm.at[src_v.at[o + 1]],
                                     rows_n, sem_n)
                pltpu.make_async_copy(x_hbm.at[src_v.at[o]],
                                      rows_c, sem_c).wait()
                pltpu.sync_copy(rows_c, acc_s.at[dst_v.at[o]], add=True)
                pltpu.sync_copy(ones_v, deg_s.at[dst_v.at[o]], add=True)
            return 0
        lax.fori_loop(0, n_chunks // IB, block_body, 0)
        plsc.subcore_barrier()

        pltpu.sync_copy(acc_s.at[pl.ds(base, rpt)],
                        acc_out.at[cid, pl.ds(base, rpt)])
        pltpu.sync_copy(deg_s.at[pl.ds(base, rpt)],
                        deg_out.at[cid, pl.ds(base, rpt)])

        @pl.when(sid == NS - 1)
        def _read_tail():
            pltpu.sync_copy(acc_s.at[pl.ds(NS * rpt, tail)],
                            acc_out.at[cid, pl.ds(NS * rpt, tail)])
            pltpu.sync_copy(deg_s.at[pl.ds(NS * rpt, tail)],
                            deg_out.at[cid, pl.ds(NS * rpt, tail)])

    return agg_kernel(x, src3, dst3, zacc, zdeg, ones8)


def _tc_fused(acc2, deg2, x, batch3, wcat, bl2, wlin_t, blin2,
              n_nodes, d_feat, h_feat, f_out, n_graphs, blk):
    nsteps = n_nodes // blk
    two_h = 2 * h_feat

    def body(acc_ref, deg_ref, x_ref, batch_ref, wcat_ref, bl_ref,
             wlin_ref, blin_ref, out_ref, max_s, sum_s, cnt_s):
        i = pl.program_id(0)

        @pl.when(i == 0)
        def _init():
            max_s[...] = jnp.full((n_graphs, h_feat), -jnp.inf, jnp.float32)
            sum_s[...] = jnp.zeros((n_graphs, h_feat), jnp.float32)
            cnt_s[...] = jnp.zeros((n_graphs, 1), jnp.float32)

        a = acc_ref[0] + acc_ref[1]
        d = deg_ref[0, :, 0:1] + deg_ref[1, :, 0:1]
        agg = a / jnp.maximum(d, 1.0)
        cat = jnp.concatenate([agg, x_ref[...]], axis=1)
        h = lax.dot_general(cat, wcat_ref[...], (((1,), (0,)), ((), ())),
                            preferred_element_type=jnp.float32)
        h = jnp.maximum(h + bl_ref[...], 0.0)

        b2 = batch_ref[0]
        gids = lax.broadcasted_iota(jnp.int32, (1, n_graphs), 1)
        onehot = (b2 == gids).astype(jnp.float32)
        sum_s[...] += lax.dot_general(onehot, h, (((0,), (0,)), ((), ())),
                                      preferred_element_type=jnp.float32)
        ones_col = jnp.ones((blk, 1), jnp.float32)
        cnt_s[...] += lax.dot_general(onehot, ones_col,
                                      (((0,), (0,)), ((), ())),
                                      preferred_element_type=jnp.float32)

        for g in range(n_graphs):
            mg = b2 == g
            hb = jnp.where(mg, h, -jnp.inf)
            mx = jnp.max(hb, axis=0, keepdims=True)
            max_s[g:g + 1, :] = jnp.maximum(max_s[g:g + 1, :], mx)

        @pl.when(i == nsteps - 1)
        def _final():
            xm = max_s[...]
            xm = jnp.where(jnp.isfinite(xm), xm, 0.0)
            mean = sum_s[...] / jnp.maximum(cnt_s[...], 1.0)
            pooled = jnp.concatenate([xm, mean], axis=1)
            out_ref[...] = lax.dot_general(
                pooled, wlin_ref[...], (((1,), (0,)), ((), ())),
                preferred_element_type=jnp.float32) + blin_ref[...]

    return pl.pallas_call(
        body,
        grid=(nsteps,),
        in_specs=[
            pl.BlockSpec((NC, blk, d_feat), lambda i: (0, i, 0)),
            pl.BlockSpec((NC, blk, DEGW), lambda i: (0, i, 0)),
            pl.BlockSpec((blk, d_feat), lambda i: (i, 0)),
            pl.BlockSpec((1, blk, 1), lambda i: (i, 0, 0)),
            pl.BlockSpec((two_h, h_feat), lambda i: (0, 0)),
            pl.BlockSpec((1, h_feat), lambda i: (0, 0)),
            pl.BlockSpec((two_h, f_out), lambda i: (0, 0)),
            pl.BlockSpec((1, f_out), lambda i: (0, 0)),
        ],
        out_specs=pl.BlockSpec((n_graphs, f_out), lambda i: (0, 0)),
        out_shape=jax.ShapeDtypeStruct((n_graphs, f_out), jnp.float32),
        scratch_shapes=[
            pltpu.VMEM((n_graphs, h_feat), jnp.float32),
            pltpu.VMEM((n_graphs, h_feat), jnp.float32),
            pltpu.VMEM((n_graphs, 1), jnp.float32),
        ],
        compiler_params=pltpu.CompilerParams(
            dimension_semantics=("arbitrary",)),
    )(acc2, deg2, x, batch3, wcat, bl2, wlin_t, blin2)


def kernel(x, edge_index, batch, W_l, b_l, W_r, W_lin, b_lin):
    n_nodes, d_feat = x.shape
    n_edges = edge_index.shape[1]
    h_feat = W_l.shape[0]
    f_out = W_lin.shape[0]
    n_graphs = 64
    n_chunks = n_edges // (NW * CHUNK)

    src3 = edge_index[0].reshape(NW, n_chunks, CHUNK)
    dst3 = edge_index[1].reshape(NW, n_chunks, CHUNK)

    acc2, deg2 = _sc_aggregate(x, src3, dst3, n_nodes, d_feat, n_chunks)

    blk = 1000
    batch3 = batch.astype(jnp.int32).reshape(n_nodes // blk, blk, 1)
    wcat = jnp.concatenate([W_l, W_r], axis=1).T
    wlin_t = W_lin.T
    bl2 = b_l.reshape(1, h_feat)
    blin2 = b_lin.reshape(1, f_out)

    return _tc_fused(acc2, deg2, x, batch3, wcat, bl2, wlin_t, blin2,
                     n_nodes, d_feat, h_feat, f_out, n_graphs, blk)


# speedup vs baseline: 2.2231x; 1.2996x over previous
import functools

import jax
import jax.numpy as jnp
from jax import lax
from jax.experimental import pallas as pl
from jax.experimental.pallas import tpu as pltpu
from jax.experimental.pallas import tpu_sc as plsc

NC, NS, L = 2, 16, 16
NW = NC * NS
CHUNK = 125
DEGW = 8
IB = 8


def _sc_aggregate(x, src3, dst3, n_nodes, d_feat, n_chunks):
    rpt = (n_nodes // NS) & ~7
    tail = n_nodes - NS * rpt
    assert tail % 8 == 0 and tail <= rpt
    assert n_chunks % IB == 0
    mesh = plsc.VectorSubcoreMesh(core_axis_name="c", subcore_axis_name="s")

    zacc = jnp.zeros((rpt, d_feat), jnp.float32)
    zdeg = jnp.zeros((rpt, DEGW), jnp.float32)
    ones8 = jnp.ones((CHUNK, DEGW), jnp.float32)

    @functools.partial(
        pl.kernel,
        mesh=mesh,
        out_type=[
            pltpu.HBM((NC, n_nodes, d_feat), jnp.float32),
            pltpu.HBM((NC, n_nodes, DEGW), jnp.float32),
        ],
        scratch_types=[
            pltpu.VMEM((IB, CHUNK), jnp.int32),
            pltpu.VMEM((IB, CHUNK), jnp.int32),
            pltpu.VMEM((CHUNK, d_feat), jnp.float32),
            pltpu.VMEM((CHUNK, d_feat), jnp.float32),
            pltpu.VMEM((CHUNK, DEGW), jnp.float32),
            pltpu.VMEM_SHARED((n_nodes, d_feat), jnp.float32),
            pltpu.VMEM_SHARED((n_nodes, DEGW), jnp.float32),
            pltpu.SemaphoreType.DMA,
            pltpu.SemaphoreType.DMA,
        ],
        compiler_params=pltpu.CompilerParams(use_tc_tiling_on_sc=False),
    )
    def agg_kernel(x_hbm, src_hbm, dst_hbm, zacc_hbm, zdeg_hbm, ones_hbm,
                   acc_out, deg_out,
                   src_v, dst_v, rows_a, rows_b, ones_v, acc_s, deg_s,
                   sem_a, sem_b):
        cid = lax.axis_index("c")
        sid = lax.axis_index("s")
        wid = sid * NC + cid
        base = sid * rpt

        pltpu.sync_copy(ones_hbm, ones_v)

        pltpu.sync_copy(zacc_hbm, acc_s.at[pl.ds(base, rpt)])
        pltpu.sync_copy(zdeg_hbm, deg_s.at[pl.ds(base, rpt)])

        @pl.when(sid == NS - 1)
        def _zero_tail():
            pltpu.sync_copy(zacc_hbm.at[pl.ds(0, tail)],
                            acc_s.at[pl.ds(NS * rpt, tail)])
            pltpu.sync_copy(zdeg_hbm.at[pl.ds(0, tail)],
                            deg_s.at[pl.ds(NS * rpt, tail)])
        plsc.subcore_barrier()

        bufs = [(rows_a, sem_a), (rows_b, sem_b)]

        def block_body(b, _):
            pltpu.sync_copy(src_hbm.at[wid, pl.ds(b * IB, IB)], src_v)
            pltpu.sync_copy(dst_hbm.at[wid, pl.ds(b * IB, IB)], dst_v)

            pltpu.async_copy(x_hbm.at[src_v.at[0]], rows_a, sem_a)
            for o in range(IB):
                rows_c, sem_c = bufs[o % 2]
                rows_n, sem_n = bufs[(o + 1) % 2]
                if o + 1 < IB:
                    pltpu.async_copy(x_hbm.at[src_v.at[o + 1]],
                                     rows_n, sem_n)
                pltpu.make_async_copy(x_hbm.at[src_v.at[o]],
                                      rows_c, sem_c).wait()
                pltpu.sync_copy(rows_c, acc_s.at[dst_v.at[o]], add=True)
                pltpu.sync_copy(ones_v, deg_s.at[dst_v.at[o]], add=True)
            return 0
        lax.fori_loop(0, n_chunks // IB, block_body, 0)
        plsc.subcore_barrier()

        pltpu.sync_copy(acc_s.at[pl.ds(base, rpt)],
                        acc_out.at[cid, pl.ds(base, rpt)])
        pltpu.sync_copy(deg_s.at[pl.ds(base, rpt)],
                        deg_out.at[cid, pl.ds(base, rpt)])

        @pl.when(sid == NS - 1)
        def _read_tail():
            pltpu.sync_copy(acc_s.at[pl.ds(NS * rpt, tail)],
                            acc_out.at[cid, pl.ds(NS * rpt, tail)])
            pltpu.sync_copy(deg_s.at[pl.ds(NS * rpt, tail)],
                            deg_out.at[cid, pl.ds(NS * rpt, tail)])

    return agg_kernel(x, src3, dst3, zacc, zdeg, ones8)


def _tc_fused(glo, ghi, acc2, deg2, x, batch3, wcat, bl2, wlin_t, blin2,
              n_nodes, d_feat, h_feat, f_out, n_graphs, blk):
    nsteps = n_nodes // blk
    two_h = 2 * h_feat

    def body(glo_ref, ghi_ref, acc_ref, deg_ref, x_ref, batch_ref,
             wcat_ref, bl_ref, wlin_ref, blin_ref, out_ref,
             max_s, sum_s, cnt_s):
        i = pl.program_id(0)

        @pl.when(i == 0)
        def _init():
            max_s[...] = jnp.full((n_graphs, h_feat), -jnp.inf, jnp.float32)
            sum_s[...] = jnp.zeros((n_graphs, h_feat), jnp.float32)
            cnt_s[...] = jnp.zeros((n_graphs, 1), jnp.float32)

        a = acc_ref[0] + acc_ref[1]
        d = deg_ref[0, :, 0:1] + deg_ref[1, :, 0:1]
        agg = a / jnp.maximum(d, 1.0)
        cat = jnp.concatenate([agg, x_ref[...]], axis=1)
        h = lax.dot_general(cat, wcat_ref[...], (((1,), (0,)), ((), ())),
                            preferred_element_type=jnp.float32)
        h = jnp.maximum(h + bl_ref[...], 0.0)

        b2 = batch_ref[0]
        gids = lax.broadcasted_iota(jnp.int32, (1, n_graphs), 1)
        onehot = (b2 == gids).astype(jnp.float32)
        sum_s[...] += lax.dot_general(onehot, h, (((0,), (0,)), ((), ())),
                                      preferred_element_type=jnp.float32)
        ones_col = jnp.ones((blk, 1), jnp.float32)
        cnt_s[...] += lax.dot_general(onehot, ones_col,
                                      (((0,), (0,)), ((), ())),
                                      preferred_element_type=jnp.float32)

        lo = glo_ref[0, 0, 0]
        hi = ghi_ref[0, 0, 0]
        for g in range(n_graphs):
            @pl.when(jnp.logical_and(g >= lo, g <= hi))
            def _masked_max(g=g):
                mg = b2 == g
                hb = jnp.where(mg, h, -jnp.inf)
                mx = jnp.max(hb, axis=0, keepdims=True)
                max_s[g:g + 1, :] = jnp.maximum(max_s[g:g + 1, :], mx)

        @pl.when(i == nsteps - 1)
        def _final():
            xm = max_s[...]
            xm = jnp.where(jnp.isfinite(xm), xm, 0.0)
            mean = sum_s[...] / jnp.maximum(cnt_s[...], 1.0)
            pooled = jnp.concatenate([xm, mean], axis=1)
            out_ref[...] = lax.dot_general(
                pooled, wlin_ref[...], (((1,), (0,)), ((), ())),
                preferred_element_type=jnp.float32) + blin_ref[...]

    return pl.pallas_call(
        body,
        grid=(nsteps,),
        in_specs=[
            pl.BlockSpec((1, 1, 1), lambda i: (i, 0, 0),
                         memory_space=pltpu.SMEM),
            pl.BlockSpec((1, 1, 1), lambda i: (i, 0, 0),
                         memory_space=pltpu.SMEM),
            pl.BlockSpec((NC, blk, d_feat), lambda i: (0, i, 0)),
            pl.BlockSpec((NC, blk, DEGW), lambda i: (0, i, 0)),
            pl.BlockSpec((blk, d_feat), lambda i: (i, 0)),
            pl.BlockSpec((1, blk, 1), lambda i: (i, 0, 0)),
            pl.BlockSpec((two_h, h_feat), lambda i: (0, 0)),
            pl.BlockSpec((1, h_feat), lambda i: (0, 0)),
            pl.BlockSpec((two_h, f_out), lambda i: (0, 0)),
            pl.BlockSpec((1, f_out), lambda i: (0, 0)),
        ],
        out_specs=pl.BlockSpec((n_graphs, f_out), lambda i: (0, 0)),
        out_shape=jax.ShapeDtypeStruct((n_graphs, f_out), jnp.float32),
        scratch_shapes=[
            pltpu.VMEM((n_graphs, h_feat), jnp.float32),
            pltpu.VMEM((n_graphs, h_feat), jnp.float32),
            pltpu.VMEM((n_graphs, 1), jnp.float32),
        ],
        compiler_params=pltpu.CompilerParams(
            dimension_semantics=("arbitrary",)),
    )(glo, ghi, acc2, deg2, x, batch3, wcat, bl2, wlin_t, blin2)


def kernel(x, edge_index, batch, W_l, b_l, W_r, W_lin, b_lin):
    n_nodes, d_feat = x.shape
    n_edges = edge_index.shape[1]
    h_feat = W_l.shape[0]
    f_out = W_lin.shape[0]
    n_graphs = 64
    n_chunks = n_edges // (NW * CHUNK)

    src3 = edge_index[0].reshape(NW, n_chunks, CHUNK)
    dst3 = edge_index[1].reshape(NW, n_chunks, CHUNK)

    acc2, deg2 = _sc_aggregate(x, src3, dst3, n_nodes, d_feat, n_chunks)

    blk = 1000
    batch2 = batch.astype(jnp.int32).reshape(n_nodes // blk, blk)
    batch3 = batch2.reshape(n_nodes // blk, blk, 1)
    glo = batch2[:, 0:1].reshape(-1, 1, 1)
    ghi = batch2[:, blk - 1:blk].reshape(-1, 1, 1)
    wcat = jnp.concatenate([W_l, W_r], axis=1).T
    wlin_t = W_lin.T
    bl2 = b_l.reshape(1, h_feat)
    blin2 = b_lin.reshape(1, f_out)

    return _tc_fused(glo, ghi, acc2, deg2, x, batch3, wcat, bl2,
                     wlin_t, blin2,
                     n_nodes, d_feat, h_feat, f_out, n_graphs, blk)


# speedup vs baseline: 2.5043x; 1.1265x over previous
import functools

import jax
import jax.numpy as jnp
from jax import lax
from jax.experimental import pallas as pl
from jax.experimental.pallas import tpu as pltpu
from jax.experimental.pallas import tpu_sc as plsc

NC, NS, L = 2, 16, 16
NW = NC * NS
CHUNK = 125
DEGW = 8
IB = 8


def _sc_aggregate(x, src3, dst3, n_nodes, d_feat, n_chunks):
    rpt = (n_nodes // NS) & ~7
    tail = n_nodes - NS * rpt
    assert tail % 8 == 0 and tail <= rpt
    assert n_chunks % IB == 0
    mesh = plsc.VectorSubcoreMesh(core_axis_name="c", subcore_axis_name="s")

    zacc = jnp.zeros((rpt, d_feat), jnp.float32)
    zdeg = jnp.zeros((rpt, DEGW), jnp.float32)
    ones8 = jnp.ones((CHUNK, DEGW), jnp.float32)

    @functools.partial(
        pl.kernel,
        mesh=mesh,
        out_type=[
            pltpu.HBM((NC, n_nodes, d_feat), jnp.float32),
            pltpu.HBM((NC, n_nodes, DEGW), jnp.float32),
        ],
        scratch_types=[
            pltpu.VMEM((2, IB, CHUNK), jnp.int32),
            pltpu.VMEM((2, IB, CHUNK), jnp.int32),
            pltpu.VMEM((CHUNK, d_feat), jnp.float32),
            pltpu.VMEM((CHUNK, d_feat), jnp.float32),
            pltpu.VMEM((CHUNK, DEGW), jnp.float32),
            pltpu.VMEM_SHARED((n_nodes, d_feat), jnp.float32),
            pltpu.VMEM_SHARED((n_nodes, DEGW), jnp.float32),
            pltpu.SemaphoreType.DMA,
            pltpu.SemaphoreType.DMA,
            pltpu.SemaphoreType.DMA,
        ],
        compiler_params=pltpu.CompilerParams(use_tc_tiling_on_sc=False),
    )
    def agg_kernel(x_hbm, src_hbm, dst_hbm, zacc_hbm, zdeg_hbm, ones_hbm,
                   acc_out, deg_out,
                   src_v, dst_v, rows_a, rows_b, ones_v, acc_s, deg_s,
                   sem_a, sem_b, isem):
        cid = lax.axis_index("c")
        sid = lax.axis_index("s")
        wid = sid * NC + cid
        base = sid * rpt

        pltpu.sync_copy(ones_hbm, ones_v)

        pltpu.sync_copy(zacc_hbm, acc_s.at[pl.ds(base, rpt)])
        pltpu.sync_copy(zdeg_hbm, deg_s.at[pl.ds(base, rpt)])

        @pl.when(sid == NS - 1)
        def _zero_tail():
            pltpu.sync_copy(zacc_hbm.at[pl.ds(0, tail)],
                            acc_s.at[pl.ds(NS * rpt, tail)])
            pltpu.sync_copy(zdeg_hbm.at[pl.ds(0, tail)],
                            deg_s.at[pl.ds(NS * rpt, tail)])
        plsc.subcore_barrier()

        bufs = [(rows_a, sem_a), (rows_b, sem_b)]
        n_blocks = n_chunks // IB

        pltpu.sync_copy(src_hbm.at[wid, pl.ds(0, IB)], src_v.at[0])
        pltpu.sync_copy(dst_hbm.at[wid, pl.ds(0, IB)], dst_v.at[0])
        if n_blocks > 1:
            pltpu.async_copy(src_hbm.at[wid, pl.ds(IB, IB)],
                             src_v.at[1], isem)
            pltpu.async_copy(dst_hbm.at[wid, pl.ds(IB, IB)],
                             dst_v.at[1], isem)
        pltpu.async_copy(x_hbm.at[src_v.at[0, 0]], rows_a, sem_a)

        def block_body(b, _):
            pid = lax.rem(b, 2)
            nid = 1 - pid
            for o in range(IB):
                rows_c, sem_c = bufs[o % 2]
                rows_n, sem_n = bufs[(o + 1) % 2]
                if o + 1 < IB:
                    pltpu.async_copy(x_hbm.at[src_v.at[pid, o + 1]],
                                     rows_n, sem_n)
                else:
                    @pl.when(b + 1 < n_blocks)
                    def _bridge():
                        pltpu.make_async_copy(
                            src_hbm.at[wid, pl.ds((b + 1) * IB, IB)],
                            src_v.at[nid], isem).wait()
                        pltpu.make_async_copy(
                            dst_hbm.at[wid, pl.ds((b + 1) * IB, IB)],
                            dst_v.at[nid], isem).wait()
                        pltpu.async_copy(x_hbm.at[src_v.at[nid, 0]],
                                         rows_n, sem_n)
                pltpu.make_async_copy(x_hbm.at[src_v.at[pid, o]],
                                      rows_c, sem_c).wait()
                pltpu.sync_copy(rows_c, acc_s.at[dst_v.at[pid, o]],
                                add=True)
                pltpu.sync_copy(ones_v, deg_s.at[dst_v.at[pid, o]],
                                add=True)
                if o == IB - 1:
                    @pl.when(b + 2 < n_blocks)
                    def _prefetch_ids():
                        pltpu.async_copy(
                            src_hbm.at[wid, pl.ds((b + 2) * IB, IB)],
                            src_v.at[pid], isem)
                        pltpu.async_copy(
                            dst_hbm.at[wid, pl.ds((b + 2) * IB, IB)],
                            dst_v.at[pid], isem)
            return 0
        lax.fori_loop(0, n_blocks, block_body, 0)
        plsc.subcore_barrier()

        pltpu.sync_copy(acc_s.at[pl.ds(base, rpt)],
                        acc_out.at[cid, pl.ds(base, rpt)])
        pltpu.sync_copy(deg_s.at[pl.ds(base, rpt)],
                        deg_out.at[cid, pl.ds(base, rpt)])

        @pl.when(sid == NS - 1)
        def _read_tail():
            pltpu.sync_copy(acc_s.at[pl.ds(NS * rpt, tail)],
                            acc_out.at[cid, pl.ds(NS * rpt, tail)])
            pltpu.sync_copy(deg_s.at[pl.ds(NS * rpt, tail)],
                            deg_out.at[cid, pl.ds(NS * rpt, tail)])

    return agg_kernel(x, src3, dst3, zacc, zdeg, ones8)


def _tc_fused(glo, ghi, acc2, deg2, x, batch3, wcat, bl2, wlin_t, blin2,
              n_nodes, d_feat, h_feat, f_out, n_graphs, blk):
    nsteps = n_nodes // blk
    two_h = 2 * h_feat

    def body(glo_ref, ghi_ref, acc_ref, deg_ref, x_ref, batch_ref,
             wcat_ref, bl_ref, wlin_ref, blin_ref, out_ref,
             max_s, sum_s, cnt_s):
        i = pl.program_id(0)

        @pl.when(i == 0)
        def _init():
            max_s[...] = jnp.full((n_graphs, h_feat), -jnp.inf, jnp.float32)
            sum_s[...] = jnp.zeros((n_graphs, h_feat), jnp.float32)
            cnt_s[...] = jnp.zeros((n_graphs, 1), jnp.float32)

        a = acc_ref[0] + acc_ref[1]
        d = deg_ref[0, :, 0:1] + deg_ref[1, :, 0:1]
        agg = a / jnp.maximum(d, 1.0)
        cat = jnp.concatenate([agg, x_ref[...]], axis=1)
        h = lax.dot_general(cat, wcat_ref[...], (((1,), (0,)), ((), ())),
                            preferred_element_type=jnp.float32)
        h = jnp.maximum(h + bl_ref[...], 0.0)

        b2 = batch_ref[0]
        gids = lax.broadcasted_iota(jnp.int32, (1, n_graphs), 1)
        onehot = (b2 == gids).astype(jnp.float32)
        sum_s[...] += lax.dot_general(onehot, h, (((0,), (0,)), ((), ())),
                                      preferred_element_type=jnp.float32)
        ones_col = jnp.ones((blk, 1), jnp.float32)
        cnt_s[...] += lax.dot_general(onehot, ones_col,
                                      (((0,), (0,)), ((), ())),
                                      preferred_element_type=jnp.float32)

        lo = glo_ref[0, 0, 0]
        hi = ghi_ref[0, 0, 0]
        for g in range(n_graphs):
            @pl.when(jnp.logical_and(g >= lo, g <= hi))
            def _masked_max(g=g):
                mg = b2 == g
                hb = jnp.where(mg, h, -jnp.inf)
                mx = jnp.max(hb, axis=0, keepdims=True)
                max_s[g:g + 1, :] = jnp.maximum(max_s[g:g + 1, :], mx)

        @pl.when(i == nsteps - 1)
        def _final():
            xm = max_s[...]
            xm = jnp.where(jnp.isfinite(xm), xm, 0.0)
            mean = sum_s[...] / jnp.maximum(cnt_s[...], 1.0)
            pooled = jnp.concatenate([xm, mean], axis=1)
            out_ref[...] = lax.dot_general(
                pooled, wlin_ref[...], (((1,), (0,)), ((), ())),
                preferred_element_type=jnp.float32) + blin_ref[...]

    return pl.pallas_call(
        body,
        grid=(nsteps,),
        in_specs=[
            pl.BlockSpec((1, 1, 1), lambda i: (i, 0, 0),
                         memory_space=pltpu.SMEM),
            pl.BlockSpec((1, 1, 1), lambda i: (i, 0, 0),
                         memory_space=pltpu.SMEM),
            pl.BlockSpec((NC, blk, d_feat), lambda i: (0, i, 0)),
            pl.BlockSpec((NC, blk, DEGW), lambda i: (0, i, 0)),
            pl.BlockSpec((blk, d_feat), lambda i: (i, 0)),
            pl.BlockSpec((1, blk, 1), lambda i: (i, 0, 0)),
            pl.BlockSpec((two_h, h_feat), lambda i: (0, 0)),
            pl.BlockSpec((1, h_feat), lambda i: (0, 0)),
            pl.BlockSpec((two_h, f_out), lambda i: (0, 0)),
            pl.BlockSpec((1, f_out), lambda i: (0, 0)),
        ],
        out_specs=pl.BlockSpec((n_graphs, f_out), lambda i: (0, 0)),
        out_shape=jax.ShapeDtypeStruct((n_graphs, f_out), jnp.float32),
        scratch_shapes=[
            pltpu.VMEM((n_graphs, h_feat), jnp.float32),
            pltpu.VMEM((n_graphs, h_feat), jnp.float32),
            pltpu.VMEM((n_graphs, 1), jnp.float32),
        ],
        compiler_params=pltpu.CompilerParams(
            dimension_semantics=("arbitrary",)),
    )(glo, ghi, acc2, deg2, x, batch3, wcat, bl2, wlin_t, blin2)


def kernel(x, edge_index, batch, W_l, b_l, W_r, W_lin, b_lin):
    n_nodes, d_feat = x.shape
    n_edges = edge_index.shape[1]
    h_feat = W_l.shape[0]
    f_out = W_lin.shape[0]
    n_graphs = 64
    n_chunks = n_edges // (NW * CHUNK)

    src3 = edge_index[0].reshape(NW, n_chunks, CHUNK)
    dst3 = edge_index[1].reshape(NW, n_chunks, CHUNK)

    acc2, deg2 = _sc_aggregate(x, src3, dst3, n_nodes, d_feat, n_chunks)

    blk = 1000
    batch2 = batch.astype(jnp.int32).reshape(n_nodes // blk, blk)
    batch3 = batch2.reshape(n_nodes // blk, blk, 1)
    glo = batch2[:, 0:1].reshape(-1, 1, 1)
    ghi = batch2[:, blk - 1:blk].reshape(-1, 1, 1)
    wcat = jnp.concatenate([W_l, W_r], axis=1).T
    wlin_t = W_lin.T
    bl2 = b_l.reshape(1, h_feat)
    blin2 = b_lin.reshape(1, f_out)

    return _tc_fused(glo, ghi, acc2, deg2, x, batch3, wcat, bl2,
                     wlin_t, blin2,
                     n_nodes, d_feat, h_feat, f_out, n_graphs, blk)


# speedup vs baseline: 2.6412x; 1.0546x over previous
import functools

import jax
import jax.numpy as jnp
from jax import lax
from jax.experimental import pallas as pl
from jax.experimental.pallas import tpu as pltpu
from jax.experimental.pallas import tpu_sc as plsc

NC, NS, L = 2, 16, 16
NW = NC * NS
CHUNK = 125
DEGW = 8
IB = 8


def _sc_aggregate(x, e4, n_nodes, d_feat, n_chunks):
    rpt = (n_nodes // NS) & ~7
    tail = n_nodes - NS * rpt
    assert tail % 8 == 0 and tail <= rpt
    assert n_chunks % IB == 0
    mesh = plsc.VectorSubcoreMesh(core_axis_name="c", subcore_axis_name="s")

    zacc = jnp.zeros((rpt, d_feat), jnp.float32)
    zdeg = jnp.zeros((rpt, DEGW), jnp.float32)
    ones8 = jnp.ones((CHUNK, DEGW), jnp.float32)

    @functools.partial(
        pl.kernel,
        mesh=mesh,
        out_type=[
            pltpu.HBM((NC, n_nodes, d_feat), jnp.float32),
            pltpu.HBM((NC, n_nodes, DEGW), jnp.float32),
        ],
        scratch_types=[
            pltpu.VMEM((2, IB, CHUNK), jnp.int32),
            pltpu.VMEM((2, IB, CHUNK), jnp.int32),
            pltpu.VMEM((CHUNK, d_feat), jnp.float32),
            pltpu.VMEM((CHUNK, d_feat), jnp.float32),
            pltpu.VMEM((CHUNK, DEGW), jnp.float32),
            pltpu.VMEM_SHARED((n_nodes, d_feat), jnp.float32),
            pltpu.VMEM_SHARED((n_nodes, DEGW), jnp.float32),
            pltpu.SemaphoreType.DMA,
            pltpu.SemaphoreType.DMA,
            pltpu.SemaphoreType.DMA,
        ],
        compiler_params=pltpu.CompilerParams(use_tc_tiling_on_sc=False),
    )
    def agg_kernel(x_hbm, e_hbm, zacc_hbm, zdeg_hbm, ones_hbm,
                   acc_out, deg_out,
                   src_v, dst_v, rows_a, rows_b, ones_v, acc_s, deg_s,
                   sem_a, sem_b, isem):
        src_hbm = e_hbm.at[0]
        dst_hbm = e_hbm.at[1]
        cid = lax.axis_index("c")
        sid = lax.axis_index("s")
        wid = sid * NC + cid
        base = sid * rpt

        pltpu.sync_copy(ones_hbm, ones_v)

        pltpu.sync_copy(zacc_hbm, acc_s.at[pl.ds(base, rpt)])
        pltpu.sync_copy(zdeg_hbm, deg_s.at[pl.ds(base, rpt)])

        @pl.when(sid == NS - 1)
        def _zero_tail():
            pltpu.sync_copy(zacc_hbm.at[pl.ds(0, tail)],
                            acc_s.at[pl.ds(NS * rpt, tail)])
            pltpu.sync_copy(zdeg_hbm.at[pl.ds(0, tail)],
                            deg_s.at[pl.ds(NS * rpt, tail)])
        plsc.subcore_barrier()

        bufs = [(rows_a, sem_a), (rows_b, sem_b)]
        n_blocks = n_chunks // IB

        pltpu.sync_copy(src_hbm.at[wid, pl.ds(0, IB)], src_v.at[0])
        pltpu.sync_copy(dst_hbm.at[wid, pl.ds(0, IB)], dst_v.at[0])
        if n_blocks > 1:
            pltpu.async_copy(src_hbm.at[wid, pl.ds(IB, IB)],
                             src_v.at[1], isem)
            pltpu.async_copy(dst_hbm.at[wid, pl.ds(IB, IB)],
                             dst_v.at[1], isem)
        pltpu.async_copy(x_hbm.at[src_v.at[0, 0]], rows_a, sem_a)

        def block_body(b, _):
            pid = lax.rem(b, 2)
            nid = 1 - pid
            for o in range(IB):
                rows_c, sem_c = bufs[o % 2]
                rows_n, sem_n = bufs[(o + 1) % 2]
                if o + 1 < IB:
                    pltpu.async_copy(x_hbm.at[src_v.at[pid, o + 1]],
                                     rows_n, sem_n)
                else:
                    @pl.when(b + 1 < n_blocks)
                    def _bridge():
                        pltpu.make_async_copy(
                            src_hbm.at[wid, pl.ds((b + 1) * IB, IB)],
                            src_v.at[nid], isem).wait()
                        pltpu.make_async_copy(
                            dst_hbm.at[wid, pl.ds((b + 1) * IB, IB)],
                            dst_v.at[nid], isem).wait()
                        pltpu.async_copy(x_hbm.at[src_v.at[nid, 0]],
                                         rows_n, sem_n)
                pltpu.make_async_copy(x_hbm.at[src_v.at[pid, o]],
                                      rows_c, sem_c).wait()
                pltpu.sync_copy(rows_c, acc_s.at[dst_v.at[pid, o]],
                                add=True)
                pltpu.sync_copy(ones_v, deg_s.at[dst_v.at[pid, o]],
                                add=True)
                if o == IB - 1:
                    @pl.when(b + 2 < n_blocks)
                    def _prefetch_ids():
                        pltpu.async_copy(
                            src_hbm.at[wid, pl.ds((b + 2) * IB, IB)],
                            src_v.at[pid], isem)
                        pltpu.async_copy(
                            dst_hbm.at[wid, pl.ds((b + 2) * IB, IB)],
                            dst_v.at[pid], isem)
            return 0
        lax.fori_loop(0, n_blocks, block_body, 0)
        plsc.subcore_barrier()

        pltpu.sync_copy(acc_s.at[pl.ds(base, rpt)],
                        acc_out.at[cid, pl.ds(base, rpt)])
        pltpu.sync_copy(deg_s.at[pl.ds(base, rpt)],
                        deg_out.at[cid, pl.ds(base, rpt)])

        @pl.when(sid == NS - 1)
        def _read_tail():
            pltpu.sync_copy(acc_s.at[pl.ds(NS * rpt, tail)],
                            acc_out.at[cid, pl.ds(NS * rpt, tail)])
            pltpu.sync_copy(deg_s.at[pl.ds(NS * rpt, tail)],
                            deg_out.at[cid, pl.ds(NS * rpt, tail)])

    return agg_kernel(x, e4, zacc, zdeg, ones8)


def _tc_fused(glo, ghi, acc2, deg2, x, batch3, wcat, bl2, wlin_t, blin2,
              n_nodes, d_feat, h_feat, f_out, n_graphs, blk):
    nsteps = n_nodes // blk
    two_h = 2 * h_feat

    def body(glo_ref, ghi_ref, acc_ref, deg_ref, x_ref, batch_ref,
             wcat_ref, bl_ref, wlin_ref, blin_ref, out_ref,
             max_s, sum_s, cnt_s):
        i = pl.program_id(0)

        @pl.when(i == 0)
        def _init():
            max_s[...] = jnp.full((n_graphs, h_feat), -jnp.inf, jnp.float32)
            sum_s[...] = jnp.zeros((n_graphs, h_feat), jnp.float32)
            cnt_s[...] = jnp.zeros((n_graphs, 1), jnp.float32)

        a = acc_ref[0] + acc_ref[1]
        d = deg_ref[0, :, 0:1] + deg_ref[1, :, 0:1]
        agg = a / jnp.maximum(d, 1.0)
        cat = jnp.concatenate([agg, x_ref[...]], axis=1)
        h = lax.dot_general(cat, wcat_ref[...], (((1,), (0,)), ((), ())),
                            preferred_element_type=jnp.float32)
        h = jnp.maximum(h + bl_ref[...], 0.0)

        b2 = batch_ref[0]
        gids = lax.broadcasted_iota(jnp.int32, (1, n_graphs), 1)
        onehot = (b2 == gids).astype(jnp.float32)
        sum_s[...] += lax.dot_general(onehot, h, (((0,), (0,)), ((), ())),
                                      preferred_element_type=jnp.float32)
        ones_col = jnp.ones((blk, 1), jnp.float32)
        cnt_s[...] += lax.dot_general(onehot, ones_col,
                                      (((0,), (0,)), ((), ())),
                                      preferred_element_type=jnp.float32)

        lo = glo_ref[0, 0, 0]
        hi = ghi_ref[0, 0, 0]
        for g in range(n_graphs):
            @pl.when(jnp.logical_and(g >= lo, g <= hi))
            def _masked_max(g=g):
                mg = b2 == g
                hb = jnp.where(mg, h, -jnp.inf)
                mx = jnp.max(hb, axis=0, keepdims=True)
                max_s[g:g + 1, :] = jnp.maximum(max_s[g:g + 1, :], mx)

        @pl.when(i == nsteps - 1)
        def _final():
            xm = max_s[...]
            xm = jnp.where(jnp.isfinite(xm), xm, 0.0)
            mean = sum_s[...] / jnp.maximum(cnt_s[...], 1.0)
            pooled = jnp.concatenate([xm, mean], axis=1)
            out_ref[...] = lax.dot_general(
                pooled, wlin_ref[...], (((1,), (0,)), ((), ())),
                preferred_element_type=jnp.float32) + blin_ref[...]

    return pl.pallas_call(
        body,
        grid=(nsteps,),
        in_specs=[
            pl.BlockSpec((1, 1, 1), lambda i: (i, 0, 0),
                         memory_space=pltpu.SMEM),
            pl.BlockSpec((1, 1, 1), lambda i: (i, 0, 0),
                         memory_space=pltpu.SMEM),
            pl.BlockSpec((NC, blk, d_feat), lambda i: (0, i, 0)),
            pl.BlockSpec((NC, blk, DEGW), lambda i: (0, i, 0)),
            pl.BlockSpec((blk, d_feat), lambda i: (i, 0)),
            pl.BlockSpec((1, blk, 1), lambda i: (i, 0, 0)),
            pl.BlockSpec((two_h, h_feat), lambda i: (0, 0)),
            pl.BlockSpec((1, h_feat), lambda i: (0, 0)),
            pl.BlockSpec((two_h, f_out), lambda i: (0, 0)),
            pl.BlockSpec((1, f_out), lambda i: (0, 0)),
        ],
        out_specs=pl.BlockSpec((n_graphs, f_out), lambda i: (0, 0)),
        out_shape=jax.ShapeDtypeStruct((n_graphs, f_out), jnp.float32),
        scratch_shapes=[
            pltpu.VMEM((n_graphs, h_feat), jnp.float32),
            pltpu.VMEM((n_graphs, h_feat), jnp.float32),
            pltpu.VMEM((n_graphs, 1), jnp.float32),
        ],
        compiler_params=pltpu.CompilerParams(
            dimension_semantics=("arbitrary",)),
    )(glo, ghi, acc2, deg2, x, batch3, wcat, bl2, wlin_t, blin2)


def kernel(x, edge_index, batch, W_l, b_l, W_r, W_lin, b_lin):
    n_nodes, d_feat = x.shape
    n_edges = edge_index.shape[1]
    h_feat = W_l.shape[0]
    f_out = W_lin.shape[0]
    n_graphs = 64
    n_chunks = n_edges // (NW * CHUNK)

    e4 = edge_index.reshape(2, NW, n_chunks, CHUNK)

    acc2, deg2 = _sc_aggregate(x, e4, n_nodes, d_feat, n_chunks)

    blk = 1000
    batch2 = batch.astype(jnp.int32).reshape(n_nodes // blk, blk)
    batch3 = batch2.reshape(n_nodes // blk, blk, 1)
    glo = batch2[:, 0:1].reshape(-1, 1, 1)
    ghi = batch2[:, blk - 1:blk].reshape(-1, 1, 1)
    wcat = jnp.concatenate([W_l, W_r], axis=1).T
    wlin_t = W_lin.T
    bl2 = b_l.reshape(1, h_feat)
    blin2 = b_lin.reshape(1, f_out)

    return _tc_fused(glo, ghi, acc2, deg2, x, batch3, wcat, bl2,
                     wlin_t, blin2,
                     n_nodes, d_feat, h_feat, f_out, n_graphs, blk)


# speedup vs baseline: 2.6829x; 1.0158x over previous
import functools

import jax
import jax.numpy as jnp
from jax import lax
from jax.experimental import pallas as pl
from jax.experimental.pallas import tpu as pltpu
from jax.experimental.pallas import tpu_sc as plsc

NC, NS, L = 2, 16, 16
NW = NC * NS
CHUNK = 125
DEGW = 8
IB = 8


def _sc_aggregate(x, e4, n_nodes, d_feat, n_chunks):
    rpt = (n_nodes // NS) & ~7
    tail = n_nodes - NS * rpt
    assert tail % 8 == 0 and tail <= rpt
    assert n_chunks % IB == 0
    mesh = plsc.VectorSubcoreMesh(core_axis_name="c", subcore_axis_name="s")

    zacc = jnp.zeros((rpt, d_feat), jnp.float32)
    zdeg = jnp.zeros((rpt, DEGW), jnp.float32)
    ones8 = jnp.ones((CHUNK, DEGW), jnp.float32)

    @functools.partial(
        pl.kernel,
        mesh=mesh,
        out_type=[
            pltpu.HBM((NC, n_nodes, d_feat), jnp.float32),
            pltpu.HBM((NC, n_nodes, DEGW), jnp.float32),
        ],
        scratch_types=[
            pltpu.VMEM((2, IB, CHUNK), jnp.int32),
            pltpu.VMEM((2, IB, CHUNK), jnp.int32),
            pltpu.VMEM((CHUNK, d_feat), jnp.float32),
            pltpu.VMEM((CHUNK, d_feat), jnp.float32),
            pltpu.VMEM((CHUNK, DEGW), jnp.float32),
            pltpu.VMEM_SHARED((n_nodes, d_feat), jnp.float32),
            pltpu.VMEM_SHARED((n_nodes, DEGW), jnp.float32),
            pltpu.SemaphoreType.DMA,
            pltpu.SemaphoreType.DMA,
            pltpu.SemaphoreType.DMA,
            pltpu.SemaphoreType.DMA,
            pltpu.SemaphoreType.DMA,
        ],
        compiler_params=pltpu.CompilerParams(use_tc_tiling_on_sc=False),
    )
    def agg_kernel(x_hbm, e_hbm, zacc_hbm, zdeg_hbm, ones_hbm,
                   acc_out, deg_out,
                   src_v, dst_v, rows_a, rows_b, ones_v, acc_s, deg_s,
                   sem_a, sem_b, isem, psem, zsem):
        src_hbm = e_hbm.at[0]
        dst_hbm = e_hbm.at[1]
        cid = lax.axis_index("c")
        sid = lax.axis_index("s")
        wid = sid * NC + cid
        base = sid * rpt

        bufs = [(rows_a, sem_a), (rows_b, sem_b)]
        n_blocks = n_chunks // IB

        pltpu.async_copy(src_hbm.at[wid, pl.ds(0, IB)], src_v.at[0], psem)
        pltpu.async_copy(dst_hbm.at[wid, pl.ds(0, IB)], dst_v.at[0], psem)
        if n_blocks > 1:
            pltpu.async_copy(src_hbm.at[wid, pl.ds(IB, IB)],
                             src_v.at[1], isem)
            pltpu.async_copy(dst_hbm.at[wid, pl.ds(IB, IB)],
                             dst_v.at[1], isem)
        pltpu.sync_copy(ones_hbm, ones_v)

        pltpu.async_copy(zacc_hbm, acc_s.at[pl.ds(base, rpt)], zsem)
        pltpu.async_copy(zdeg_hbm, deg_s.at[pl.ds(base, rpt)], zsem)

        @pl.when(sid == NS - 1)
        def _zero_tail():
            pltpu.sync_copy(zacc_hbm.at[pl.ds(0, tail)],
                            acc_s.at[pl.ds(NS * rpt, tail)])
            pltpu.sync_copy(zdeg_hbm.at[pl.ds(0, tail)],
                            deg_s.at[pl.ds(NS * rpt, tail)])

        pltpu.make_async_copy(src_hbm.at[wid, pl.ds(0, IB)],
                              src_v.at[0], psem).wait()
        pltpu.make_async_copy(dst_hbm.at[wid, pl.ds(0, IB)],
                              dst_v.at[0], psem).wait()
        pltpu.async_copy(x_hbm.at[src_v.at[0, 0]], rows_a, sem_a)
        pltpu.make_async_copy(zacc_hbm, acc_s.at[pl.ds(base, rpt)],
                              zsem).wait()
        pltpu.make_async_copy(zdeg_hbm, deg_s.at[pl.ds(base, rpt)],
                              zsem).wait()
        plsc.subcore_barrier()

        def block_body(b, _):
            pid = lax.rem(b, 2)
            nid = 1 - pid
            for o in range(IB):
                rows_c, sem_c = bufs[o % 2]
                rows_n, sem_n = bufs[(o + 1) % 2]
                if o + 1 < IB:
                    pltpu.async_copy(x_hbm.at[src_v.at[pid, o + 1]],
                                     rows_n, sem_n)
                else:
                    @pl.when(b + 1 < n_blocks)
                    def _bridge():
                        pltpu.make_async_copy(
                            src_hbm.at[wid, pl.ds((b + 1) * IB, IB)],
                            src_v.at[nid], isem).wait()
                        pltpu.make_async_copy(
                            dst_hbm.at[wid, pl.ds((b + 1) * IB, IB)],
                            dst_v.at[nid], isem).wait()
                        pltpu.async_copy(x_hbm.at[src_v.at[nid, 0]],
                                         rows_n, sem_n)
                pltpu.make_async_copy(x_hbm.at[src_v.at[pid, o]],
                                      rows_c, sem_c).wait()
                pltpu.sync_copy(rows_c, acc_s.at[dst_v.at[pid, o]],
                                add=True)
                pltpu.sync_copy(ones_v, deg_s.at[dst_v.at[pid, o]],
                                add=True)
                if o == IB - 1:
                    @pl.when(b + 2 < n_blocks)
                    def _prefetch_ids():
                        pltpu.async_copy(
                            src_hbm.at[wid, pl.ds((b + 2) * IB, IB)],
                            src_v.at[pid], isem)
                        pltpu.async_copy(
                            dst_hbm.at[wid, pl.ds((b + 2) * IB, IB)],
                            dst_v.at[pid], isem)
            return 0
        lax.fori_loop(0, n_blocks, block_body, 0)
        plsc.subcore_barrier()

        pltpu.sync_copy(acc_s.at[pl.ds(base, rpt)],
                        acc_out.at[cid, pl.ds(base, rpt)])
        pltpu.sync_copy(deg_s.at[pl.ds(base, rpt)],
                        deg_out.at[cid, pl.ds(base, rpt)])

        @pl.when(sid == NS - 1)
        def _read_tail():
            pltpu.sync_copy(acc_s.at[pl.ds(NS * rpt, tail)],
                            acc_out.at[cid, pl.ds(NS * rpt, tail)])
            pltpu.sync_copy(deg_s.at[pl.ds(NS * rpt, tail)],
                            deg_out.at[cid, pl.ds(NS * rpt, tail)])

    return agg_kernel(x, e4, zacc, zdeg, ones8)


def _tc_fused(glo, ghi, acc2, deg2, x, batch3, wcat, bl2, wlin_t, blin2,
              n_nodes, d_feat, h_feat, f_out, n_graphs, blk):
    nsteps = n_nodes // blk
    two_h = 2 * h_feat

    def body(glo_ref, ghi_ref, acc_ref, deg_ref, x_ref, batch_ref,
             wcat_ref, bl_ref, wlin_ref, blin_ref, out_ref,
             max_s, sum_s, cnt_s):
        i = pl.program_id(0)

        @pl.when(i == 0)
        def _init():
            max_s[...] = jnp.full((n_graphs, h_feat), -jnp.inf, jnp.float32)
            sum_s[...] = jnp.zeros((n_graphs, h_feat), jnp.float32)
            cnt_s[...] = jnp.zeros((n_graphs, 1), jnp.float32)

        a = acc_ref[0] + acc_ref[1]
        d = deg_ref[0, :, 0:1] + deg_ref[1, :, 0:1]
        agg = a / jnp.maximum(d, 1.0)
        cat = jnp.concatenate([agg, x_ref[...]], axis=1)
        h = lax.dot_general(cat, wcat_ref[...], (((1,), (0,)), ((), ())),
                            preferred_element_type=jnp.float32)
        h = jnp.maximum(h + bl_ref[...], 0.0)

        b2 = batch_ref[0]
        gids = lax.broadcasted_iota(jnp.int32, (1, n_graphs), 1)
        onehot = (b2 == gids).astype(jnp.float32)
        sum_s[...] += lax.dot_general(onehot, h, (((0,), (0,)), ((), ())),
                                      preferred_element_type=jnp.float32)
        ones_col = jnp.ones((blk, 1), jnp.float32)
        cnt_s[...] += lax.dot_general(onehot, ones_col,
                                      (((0,), (0,)), ((), ())),
                                      preferred_element_type=jnp.float32)

        lo = glo_ref[0, 0, 0]
        hi = ghi_ref[0, 0, 0]
        for g in range(n_graphs):
            @pl.when(jnp.logical_and(g >= lo, g <= hi))
            def _masked_max(g=g):
                mg = b2 == g
                hb = jnp.where(mg, h, -jnp.inf)
                mx = jnp.max(hb, axis=0, keepdims=True)
                max_s[g:g + 1, :] = jnp.maximum(max_s[g:g + 1, :], mx)

        @pl.when(i == nsteps - 1)
        def _final():
            xm = max_s[...]
            xm = jnp.where(jnp.isfinite(xm), xm, 0.0)
            mean = sum_s[...] / jnp.maximum(cnt_s[...], 1.0)
            pooled = jnp.concatenate([xm, mean], axis=1)
            out_ref[...] = lax.dot_general(
                pooled, wlin_ref[...], (((1,), (0,)), ((), ())),
                preferred_element_type=jnp.float32) + blin_ref[...]

    return pl.pallas_call(
        body,
        grid=(nsteps,),
        in_specs=[
            pl.BlockSpec((1, 1, 1), lambda i: (i, 0, 0),
                         memory_space=pltpu.SMEM),
            pl.BlockSpec((1, 1, 1), lambda i: (i, 0, 0),
                         memory_space=pltpu.SMEM),
            pl.BlockSpec((NC, blk, d_feat), lambda i: (0, i, 0)),
            pl.BlockSpec((NC, blk, DEGW), lambda i: (0, i, 0)),
            pl.BlockSpec((blk, d_feat), lambda i: (i, 0)),
            pl.BlockSpec((1, blk, 1), lambda i: (i, 0, 0)),
            pl.BlockSpec((two_h, h_feat), lambda i: (0, 0)),
            pl.BlockSpec((1, h_feat), lambda i: (0, 0)),
            pl.BlockSpec((two_h, f_out), lambda i: (0, 0)),
            pl.BlockSpec((1, f_out), lambda i: (0, 0)),
        ],
        out_specs=pl.BlockSpec((n_graphs, f_out), lambda i: (0, 0)),
        out_shape=jax.ShapeDtypeStruct((n_graphs, f_out), jnp.float32),
        scratch_shapes=[
            pltpu.VMEM((n_graphs, h_feat), jnp.float32),
            pltpu.VMEM((n_graphs, h_feat), jnp.float32),
            pltpu.VMEM((n_graphs, 1), jnp.float32),
        ],
        compiler_params=pltpu.CompilerParams(
            dimension_semantics=("arbitrary",)),
    )(glo, ghi, acc2, deg2, x, batch3, wcat, bl2, wlin_t, blin2)


def kernel(x, edge_index, batch, W_l, b_l, W_r, W_lin, b_lin):
    n_nodes, d_feat = x.shape
    n_edges = edge_index.shape[1]
    h_feat = W_l.shape[0]
    f_out = W_lin.shape[0]
    n_graphs = 64
    n_chunks = n_edges // (NW * CHUNK)

    e4 = edge_index.reshape(2, NW, n_chunks, CHUNK)

    acc2, deg2 = _sc_aggregate(x, e4, n_nodes, d_feat, n_chunks)

    blk = 1000
    batch2 = batch.astype(jnp.int32).reshape(n_nodes // blk, blk)
    batch3 = batch2.reshape(n_nodes // blk, blk, 1)
    glo = batch2[:, 0:1].reshape(-1, 1, 1)
    ghi = batch2[:, blk - 1:blk].reshape(-1, 1, 1)
    wcat = jnp.concatenate([W_l, W_r], axis=1).T
    wlin_t = W_lin.T
    bl2 = b_l.reshape(1, h_feat)
    blin2 = b_lin.reshape(1, f_out)

    return _tc_fused(glo, ghi, acc2, deg2, x, batch3, wcat, bl2,
                     wlin_t, blin2,
                     n_nodes, d_feat, h_feat, f_out, n_graphs, blk)
